# speedup vs baseline: 1.5564x; 1.5564x over previous
import functools

import jax
import jax.numpy as jnp
from jax import lax
from jax.experimental import pallas as pl
from jax.experimental.pallas import tpu as pltpu
from jax.experimental.pallas import tpu_sc as plsc

_B = 16384
_D = 128

_info = plsc.get_sparse_core_info()
_NC, _NS = _info.num_cores, _info.num_subcores
_NW = _NC * _NS
_BPW = _B // _NW

_mesh = plsc.VectorSubcoreMesh(core_axis_name="c", subcore_axis_name="s")


@functools.partial(
    pl.kernel,
    out_type=(
        jax.ShapeDtypeStruct((_B, _D), jnp.float32),
        jax.ShapeDtypeStruct((_B, _D), jnp.float32),
    ),
    mesh=_mesh,
    scratch_types=[
        pltpu.VMEM((_BPW,), jnp.int32),
        pltpu.VMEM((_BPW,), jnp.int32),
        pltpu.VMEM((_BPW, _D), jnp.float32),
        pltpu.SemaphoreType.DMA,
    ],
)
def _gather2(X_hbm, y_hbm, W1_hbm, W2_hbm, out1_hbm, out2_hbm,
             idx1_v, idx2_v, rows_v, sem):
    wid = lax.axis_index("s") * _NC + lax.axis_index("c")
    base = wid * _BPW
    pltpu.sync_copy(X_hbm.at[pl.ds(base, _BPW)], idx1_v)
    gather1 = pltpu.async_copy(W1_hbm.at[idx1_v], rows_v, sem)
    pltpu.sync_copy(y_hbm.at[pl.ds(base, _BPW)], idx2_v)
    gather1.wait()
    pltpu.sync_copy(rows_v, out1_hbm.at[pl.ds(base, _BPW)])
    pltpu.async_copy(W2_hbm.at[idx2_v], rows_v, sem).wait()
    pltpu.sync_copy(rows_v, out2_hbm.at[pl.ds(base, _BPW)])


def kernel(X, y, W1, W2):
    return _gather2(X, y, W1, W2)
